# baseline (speedup 1.0000x reference)
import functools

import jax
import jax.numpy as jnp
from jax import lax
from jax.experimental import pallas as pl
from jax.experimental.pallas import tpu as pltpu
from jax.experimental.pallas import tpu_sc as plsc

NUM_CLASSES = 1000
N_ROWS = 16384
SEG = 26
FLAT_X = N_ROWS * SEG
FLAT_OUT = FLAT_X * NUM_CLASSES

NC, NS = 2, 16
NW = NC * NS
ROWS_PER_W = N_ROWS // NW
C = 2
XC = C * SEG
XC_PAD = 64
CHUNK = XC * NUM_CLASSES
NCHUNK = ROWS_PER_W // C

_mesh = plsc.VectorSubcoreMesh(core_axis_name="c", subcore_axis_name="s")


@functools.partial(
    pl.kernel,
    out_type=jax.ShapeDtypeStruct((FLAT_OUT,), jnp.float32),
    mesh=_mesh,
    compiler_params=pltpu.CompilerParams(needs_layout_passes=False),
    scratch_types=[
        pltpu.VMEM((2 * CHUNK,), jnp.float32),
        pltpu.VMEM((NCHUNK * XC_PAD,), jnp.int32),
        pltpu.SemaphoreType.DMA,
        pltpu.SemaphoreType.DMA,
        pltpu.SemaphoreType.DMA,
    ],
)
def _sc_onehot(xp_hbm, out_hbm, buf, xv, sem0, sem1, semx):
    wid = lax.axis_index("s") * NC + lax.axis_index("c")
    obase = wid * (ROWS_PER_W * SEG * NUM_CLASSES)

    pltpu.async_copy(
        xp_hbm.at[pl.ds(wid * NCHUNK * XC_PAD, NCHUNK * XC_PAD)], xv, semx
    ).wait()

    zeros16 = jnp.zeros((16,), jnp.float32)
    ones16 = jnp.ones((16,), jnp.float32)
    lane = lax.iota(jnp.int32, 16)
    sems = (sem0, sem1)

    @pl.loop(0, (2 * CHUNK) // 16)
    def _zero(j):
        buf[pl.ds(j * 16, 16)] = zeros16

    def scat(k, b, vals):
        for g in range(4):
            s = lane + 16 * g
            m = s < XC
            xs = xv[pl.ds(k * XC_PAD + 16 * g, 16)]
            idx = s * NUM_CLASSES + xs + b * CHUNK
            plsc.store_scatter(buf, [idx], vals, mask=m)

    @pl.loop(0, NCHUNK // 2)
    def _body(kk):
        for b in range(2):
            k = kk * 2 + b

            @pl.when(k >= 2)
            def _():
                pltpu.make_async_copy(
                    buf.at[pl.ds(b * CHUNK, CHUNK)],
                    out_hbm.at[pl.ds(obase, CHUNK)],
                    sems[b],
                ).wait()
                scat(k - 2, b, zeros16)

            scat(k, b, ones16)
            pltpu.async_copy(
                buf.at[pl.ds(b * CHUNK, CHUNK)],
                out_hbm.at[pl.ds(obase + k * CHUNK, CHUNK)],
                sems[b],
            )

    for b in range(2):
        pltpu.make_async_copy(
            buf.at[pl.ds(b * CHUNK, CHUNK)],
            out_hbm.at[pl.ds(obase, CHUNK)],
            sems[b],
        ).wait()


def kernel(x):
    xp = jnp.pad(
        x.astype(jnp.int32).reshape(FLAT_X // XC, XC), ((0, 0), (0, XC_PAD - XC))
    ).reshape(-1)
    return _sc_onehot(xp).reshape(N_ROWS, SEG, NUM_CLASSES)


# speedup vs baseline: 1.0025x; 1.0025x over previous
import functools

import jax
import jax.numpy as jnp
from jax import lax
from jax.experimental import pallas as pl
from jax.experimental.pallas import tpu as pltpu
from jax.experimental.pallas import tpu_sc as plsc

NUM_CLASSES = 1000
N_ROWS = 16384
SEG = 26

NC, NS = 2, 16
NW = NC * NS
ROWS_PER_W = N_ROWS // NW
C = 2
XC = C * SEG
XC_PAD = 64
NCHUNK = ROWS_PER_W // C

_mesh = plsc.VectorSubcoreMesh(core_axis_name="c", subcore_axis_name="s")


@functools.partial(
    pl.kernel,
    out_type=jax.ShapeDtypeStruct((N_ROWS, SEG, NUM_CLASSES), jnp.float32),
    mesh=_mesh,
    compiler_params=pltpu.CompilerParams(
        needs_layout_passes=False, use_tc_tiling_on_sc=False
    ),
    scratch_types=[
        pltpu.VMEM((C, SEG, NUM_CLASSES), jnp.float32),
        pltpu.VMEM((C, SEG, NUM_CLASSES), jnp.float32),
        pltpu.VMEM((NCHUNK * XC_PAD,), jnp.int32),
        pltpu.SemaphoreType.DMA,
        pltpu.SemaphoreType.DMA,
        pltpu.SemaphoreType.DMA,
    ],
)
def _sc_onehot(xp_hbm, out_hbm, buf0, buf1, xv, sem0, sem1, semx):
    wid = lax.axis_index("s") * NC + lax.axis_index("c")
    rbase = wid * ROWS_PER_W

    pltpu.async_copy(
        xp_hbm.at[pl.ds(wid * NCHUNK * XC_PAD, NCHUNK * XC_PAD)], xv, semx
    ).wait()

    zeros16 = jnp.zeros((16,), jnp.float32)
    ones16 = jnp.ones((16,), jnp.float32)
    lane = lax.iota(jnp.int32, 16)
    bufs = (buf0, buf1)
    sems = (sem0, sem1)

    offs = list(range(0, 16 * (NUM_CLASSES // 16), 16)) + [NUM_CLASSES - 16]

    @pl.loop(0, SEG)
    def _zero(c):
        for buf in bufs:
            for r in range(C):
                for o in offs:
                    buf[r, c, pl.ds(o, 16)] = zeros16

    def scat(k, b, vals):
        for g in range(XC_PAD // 16):
            s = lane + 16 * g
            m = s < XC
            r = (s >= SEG).astype(jnp.int32)
            c = s - SEG * r
            xs = xv[pl.ds(k * XC_PAD + 16 * g, 16)]
            plsc.store_scatter(bufs[b], [r, c, xs], vals, mask=m)

    @pl.loop(0, NCHUNK // 2)
    def _body(kk):
        for b in range(2):
            k = kk * 2 + b

            @pl.when(k >= 2)
            def _():
                pltpu.make_async_copy(
                    bufs[b], out_hbm.at[pl.ds(rbase, C)], sems[b]
                ).wait()
                scat(k - 2, b, zeros16)

            scat(k, b, ones16)
            pltpu.async_copy(
                bufs[b], out_hbm.at[pl.ds(rbase + k * C, C)], sems[b]
            )

    for b in range(2):
        pltpu.make_async_copy(
            bufs[b], out_hbm.at[pl.ds(rbase, C)], sems[b]
        ).wait()


def kernel(x):
    xp = jnp.pad(
        x.astype(jnp.int32).reshape(N_ROWS * SEG // XC, XC),
        ((0, 0), (0, XC_PAD - XC)),
    ).reshape(-1)
    return _sc_onehot(xp)


# speedup vs baseline: 2.0327x; 2.0276x over previous
import functools

import jax
import jax.numpy as jnp
from jax import lax
from jax.experimental import pallas as pl
from jax.experimental.pallas import tpu as pltpu
from jax.experimental.pallas import tpu_sc as plsc

NUM_CLASSES = 1000
N_ROWS = 16384
SEG = 26

NC, NS = 2, 16
NW = NC * NS
ROWS_PER_W = N_ROWS // NW
XC_PAD = 32

_mesh = plsc.VectorSubcoreMesh(core_axis_name="c", subcore_axis_name="s")


@functools.partial(
    pl.kernel,
    out_type=jax.ShapeDtypeStruct((N_ROWS, SEG, NUM_CLASSES), jnp.float32),
    mesh=_mesh,
    compiler_params=pltpu.CompilerParams(needs_layout_passes=False),
    scratch_types=[
        pltpu.VMEM((1, SEG, NUM_CLASSES), jnp.float32),
        pltpu.VMEM((1, SEG, NUM_CLASSES), jnp.float32),
        pltpu.VMEM((ROWS_PER_W * XC_PAD,), jnp.int32),
        pltpu.SemaphoreType.DMA,
        pltpu.SemaphoreType.DMA,
        pltpu.SemaphoreType.DMA,
    ],
)
def _sc_onehot(xp_hbm, out_hbm, buf0, buf1, xv, sem0, sem1, semx):
    wid = lax.axis_index("s") * NC + lax.axis_index("c")
    rbase = wid * ROWS_PER_W

    pltpu.async_copy(
        xp_hbm.at[pl.ds(wid * ROWS_PER_W * XC_PAD, ROWS_PER_W * XC_PAD)],
        xv,
        semx,
    ).wait()

    zeros16 = jnp.zeros((16,), jnp.float32)
    ones16 = jnp.ones((16,), jnp.float32)
    lane = lax.iota(jnp.int32, 16)
    zero16i = jnp.zeros((16,), jnp.int32)
    bufs = (buf0, buf1)
    sems = (sem0, sem1)

    offs = list(range(0, 16 * (NUM_CLASSES // 16), 16)) + [NUM_CLASSES - 16]

    @pl.loop(0, SEG)
    def _zero(c):
        for buf in bufs:
            for o in offs:
                buf[0, c, pl.ds(o, 16)] = zeros16

    def scat(k, b, vals):
        for g in range(XC_PAD // 16):
            s = lane + 16 * g
            m = s < SEG
            xs = xv[pl.ds(k * XC_PAD + 16 * g, 16)]
            plsc.store_scatter(bufs[b], [zero16i, s, xs], vals, mask=m)

    @pl.loop(0, ROWS_PER_W // 2)
    def _body(kk):
        for b in range(2):
            k = kk * 2 + b

            @pl.when(k >= 2)
            def _():
                pltpu.make_async_copy(
                    bufs[b], out_hbm.at[pl.ds(rbase, 1)], sems[b]
                ).wait()
                scat(k - 2, b, zeros16)

            scat(k, b, ones16)
            pltpu.async_copy(
                bufs[b], out_hbm.at[pl.ds(rbase + k, 1)], sems[b]
            )

    for b in range(2):
        pltpu.make_async_copy(
            bufs[b], out_hbm.at[pl.ds(rbase, 1)], sems[b]
        ).wait()


def kernel(x):
    xp = jnp.pad(x.astype(jnp.int32), ((0, 0), (0, XC_PAD - SEG))).reshape(-1)
    return _sc_onehot(xp)


# speedup vs baseline: 2.0478x; 1.0074x over previous
import functools

import jax
import jax.numpy as jnp
from jax import lax
from jax.experimental import pallas as pl
from jax.experimental.pallas import tpu as pltpu
from jax.experimental.pallas import tpu_sc as plsc

NUM_CLASSES = 1000
N_ROWS = 16384
SEG = 26

NC, NS = 2, 16
NW = NC * NS
ROWS_PER_W = N_ROWS // NW
C = 2
XC = C * SEG
XC_PAD = 64
NCHUNK = ROWS_PER_W // C

_mesh = plsc.VectorSubcoreMesh(core_axis_name="c", subcore_axis_name="s")


@functools.partial(
    pl.kernel,
    out_type=jax.ShapeDtypeStruct((N_ROWS, SEG, NUM_CLASSES), jnp.float32),
    mesh=_mesh,
    compiler_params=pltpu.CompilerParams(needs_layout_passes=False),
    scratch_types=[
        pltpu.VMEM((C, SEG, NUM_CLASSES), jnp.float32),
        pltpu.VMEM((NCHUNK * XC_PAD,), jnp.int32),
        pltpu.SemaphoreType.DMA,
        pltpu.SemaphoreType.DMA,
    ],
)
def _sc_onehot(xp_hbm, out_hbm, buf, xv, sem, semx):
    wid = lax.axis_index("s") * NC + lax.axis_index("c")
    rbase = wid * ROWS_PER_W

    pltpu.async_copy(
        xp_hbm.at[pl.ds(wid * NCHUNK * XC_PAD, NCHUNK * XC_PAD)], xv, semx
    ).wait()

    zeros16 = jnp.zeros((16,), jnp.float32)
    ones16 = jnp.ones((16,), jnp.float32)
    lane = lax.iota(jnp.int32, 16)

    offs = list(range(0, 16 * (NUM_CLASSES // 16), 16)) + [NUM_CLASSES - 16]

    @pl.loop(0, SEG)
    def _zero(c):
        for r in range(C):
            for o in offs:
                buf[r, c, pl.ds(o, 16)] = zeros16

    def scat(k, vals):
        for g in range(XC_PAD // 16):
            s = lane + 16 * g
            m = s < XC
            r = (s >= SEG).astype(jnp.int32)
            c = s - SEG * r
            xs = xv[pl.ds(k * XC_PAD + 16 * g, 16)]
            plsc.store_scatter(buf, [r, c, xs], vals, mask=m)

    @pl.loop(0, NCHUNK)
    def _body(k):
        scat(k, ones16)
        pltpu.async_copy(
            buf, out_hbm.at[pl.ds(rbase + k * C, C)], sem
        ).wait()
        scat(k, zeros16)


def kernel(x):
    xp = jnp.pad(
        x.astype(jnp.int32).reshape(N_ROWS * SEG // XC, XC),
        ((0, 0), (0, XC_PAD - XC)),
    ).reshape(-1)
    return _sc_onehot(xp)


# speedup vs baseline: 8.0146x; 3.9138x over previous
import jax
import jax.numpy as jnp
from jax.experimental import pallas as pl

NUM_CLASSES = 1000
N_ROWS = 16384
SEG = 26
BC = 8
BI = 8192


def _onehot_body(xt_ref, o_ref):
    cls0 = pl.program_id(0) * BC
    cls = cls0 + jax.lax.broadcasted_iota(jnp.int32, (SEG, BC, BI), 1)
    o_ref[...] = (xt_ref[...][:, None, :] == cls).astype(jnp.float32)


def kernel(x):
    xt = x.astype(jnp.int32).T
    out_t = pl.pallas_call(
        _onehot_body,
        grid=(NUM_CLASSES // BC, N_ROWS // BI),
        in_specs=[pl.BlockSpec((SEG, BI), lambda ci, ii: (0, ii))],
        out_specs=pl.BlockSpec((SEG, BC, BI), lambda ci, ii: (0, ci, ii)),
        out_shape=jax.ShapeDtypeStruct((SEG, NUM_CLASSES, N_ROWS), jnp.float32),
    )(xt)
    return out_t.transpose(2, 0, 1)
